# baseline (speedup 1.0000x reference)
import functools

import jax
import jax.numpy as jnp
from jax import lax
from jax.experimental import pallas as pl
from jax.experimental.pallas import tpu as pltpu

N_SPECIES = 400
N_GT = 500


def _contrast_body(spc_ref, clade_ref, out_ref):
    spc = spc_ref[...]
    clade = clade_ref[...]
    outm = 1.0 - clade
    validf = (spc > 0).astype(jnp.float32)
    dup = (spc > 1).astype(jnp.float32)

    dn = (((1,), (1,)), ((), ()))

    def mm(a, b):
        return lax.dot_general(a, b, dn, preferred_element_type=jnp.float32)

    cb = mm(spc, clade)
    co = mm(spc, outm)
    vb = mm(validf, clade)
    vo = mm(validf, outm)
    db = mm(dup, clade)
    do = mm(dup, outm)

    has = (vb > 0) & (vo > 0)
    avg_b = cb / jnp.maximum(vb, 1.0)
    avg_o = co / jnp.maximum(vo, 1.0)
    cr = avg_b / jnp.maximum(avg_o, 0.1)
    fdb = db / jnp.maximum(vb, 1.0)
    fdo = do / jnp.maximum(vo, 1.0)
    dc = fdb - fdo
    m = has.astype(jnp.float32)
    n = m.sum(axis=0)

    feats = []
    for xx in (avg_b, cr, fdb, fdo, dc):
        mu = (xx * m).sum(0) / jnp.maximum(n, 1.0)
        var = (((xx - mu[None, :]) ** 2) * m).sum(0) / jnp.maximum(n - 1.0, 1.0)
        sd = jnp.where(n > 1, jnp.sqrt(jnp.maximum(var, 0.0) + 1e-12), 0.0)
        feats.append(mu[:, None])
        feats.append(sd[:, None])
    contrast = jnp.concatenate(feats, axis=1)
    edge_ok = (clade.sum(1) > 0) & (outm.sum(1) > 0) & (n > 0)
    out_ref[...] = contrast * edge_ok[:, None].astype(jnp.float32)


def _contrast(spc2d, clade_f):
    e = clade_f.shape[0]
    return pl.pallas_call(
        _contrast_body,
        out_shape=jax.ShapeDtypeStruct((e, 10), jnp.float32),
    )(spc2d, clade_f)


def kernel(species_emb, gin_params, ln_params, species_ids, leaf_mask, batch_ids, edge_index, clade_mask, n_edges):
    sp = species_ids
    valid = leaf_mask & (sp >= 0)
    composite = batch_ids * N_SPECIES + jnp.clip(sp, 0, N_SPECIES - 1)
    ones = jnp.where(valid, 1.0, 0.0)
    sp_count = jnp.zeros((N_GT * N_SPECIES,), jnp.float32).at[composite].add(ones)
    sp_count_2d = sp_count.reshape(N_GT, N_SPECIES)
    validf = (sp_count_2d > 0).astype(jnp.float32)

    clade = clade_mask.astype(jnp.float32)
    contrast = _contrast(sp_count_2d, clade)

    emb_ids = jnp.where(sp < 0, N_SPECIES, sp)
    emb_ids = jnp.clip(emb_ids, 0, N_SPECIES)
    x = species_emb[emb_ids]
    src = edge_index[0]
    dst = edge_index[1]
    for gp, lp in zip(gin_params, ln_params):
        agg = jnp.zeros_like(x).at[dst].add(x[src])
        h = (1.0 + gp['eps']) * x + agg
        h = jnp.maximum(h @ gp['W1'] + gp['b1'], 0.0) @ gp['W2'] + gp['b2']
        x = x + h
        mu = x.mean(axis=-1, keepdims=True)
        var = x.var(axis=-1, keepdims=True)
        x = (x - mu) / jnp.sqrt(var + 1e-5) * lp['g'] + lp['b']

    EMBED_DIM = x.shape[1]
    lx = x * ones[:, None]
    pool = jnp.zeros((N_GT * N_SPECIES, EMBED_DIM), jnp.float32).at[composite].add(lx)
    mean_pool = pool / jnp.maximum(sp_count, 1.0)[:, None]
    mp = mean_pool.reshape(N_GT, N_SPECIES, EMBED_DIM)
    vmask = validf[:, :, None]
    M1 = (mp * vmask).sum(0)
    M2 = ((mp ** 2) * vmask).sum(0)
    C = validf.sum(0)
    S1 = clade @ M1
    S2 = clade @ M2
    Ne = clade @ C
    mean_e = S1 / jnp.maximum(Ne, 1.0)[:, None]
    var_e = (S2 - jnp.maximum(Ne, 1.0)[:, None] * mean_e ** 2) / jnp.maximum(Ne - 1.0, 1.0)[:, None]
    std_e = jnp.where((Ne > 1)[:, None], jnp.sqrt(jnp.maximum(var_e, 0.0) + 1e-12), 0.0)
    gin_feats = jnp.concatenate([mean_e, std_e], axis=1)

    return jnp.concatenate([gin_feats, contrast], axis=1)


# speedup vs baseline: 1.2005x; 1.2005x over previous
import functools

import jax
import jax.numpy as jnp
from jax import lax
from jax.experimental import pallas as pl
from jax.experimental.pallas import tpu as pltpu
from jax.experimental.pallas import tpu_sc as plsc

N_SPECIES = 400
N_GT = 500
EMB_PAD = 512
ROW_BLK = 2000
GT_BLK = 2


def _emb_body(ids_ref, table_ref, out_ref):
    ids = ids_ref[0, 0]
    table = table_ref[...]
    cols = lax.broadcasted_iota(jnp.int32, (ROW_BLK, EMB_PAD), 1)
    onehot = (ids[:, None] == cols).astype(jnp.float32)
    out_ref[...] = lax.dot_general(
        onehot, table, (((1,), (0,)), ((), ())),
        preferred_element_type=jnp.float32)


def _emb_lookup(emb_ids, table_pad, n_nodes):
    grid = n_nodes // ROW_BLK
    ids3 = emb_ids.reshape(grid, 1, ROW_BLK)
    return pl.pallas_call(
        _emb_body,
        grid=(grid,),
        in_specs=[
            pl.BlockSpec((1, 1, ROW_BLK), lambda i: (i, 0, 0)),
            pl.BlockSpec((EMB_PAD, 64), lambda i: (0, 0)),
        ],
        out_specs=pl.BlockSpec((ROW_BLK, 64), lambda i: (i, 0)),
        out_shape=jax.ShapeDtypeStruct((n_nodes, 64), jnp.float32),
    )(ids3, table_pad)


def _mlp_body(x_ref, agg_ref, w1_ref, b1_ref, w2_ref, b2_ref, g_ref, b_ref,
              eps_ref, out_ref):
    x = x_ref[...]
    agg = agg_ref[...]
    eps = eps_ref[0, 0]
    h = (1.0 + eps) * x + agg
    dn = (((1,), (0,)), ((), ()))
    z = lax.dot_general(h, w1_ref[...], dn, preferred_element_type=jnp.float32)
    z = jnp.maximum(z + b1_ref[0][None, :], 0.0)
    h2 = lax.dot_general(z, w2_ref[...], dn, preferred_element_type=jnp.float32)
    h2 = h2 + b2_ref[0][None, :]
    xn = x + h2
    mu = jnp.mean(xn, axis=-1, keepdims=True)
    var = jnp.mean((xn - mu) ** 2, axis=-1, keepdims=True)
    out_ref[...] = ((xn - mu) * lax.rsqrt(var + 1e-5) * g_ref[0][None, :]
                    + b_ref[0][None, :])


def _mlp_layer(x, agg, gp, lp, n_nodes):
    grid = n_nodes // ROW_BLK
    row = pl.BlockSpec((ROW_BLK, 64), lambda i: (i, 0))
    mat = pl.BlockSpec((64, 64), lambda i: (0, 0))
    vec = pl.BlockSpec((1, 64), lambda i: (0, 0))
    scl = pl.BlockSpec((1, 1), lambda i: (0, 0))
    return pl.pallas_call(
        _mlp_body,
        grid=(grid,),
        in_specs=[row, row, mat, vec, mat, vec, vec, vec, scl],
        out_specs=row,
        out_shape=jax.ShapeDtypeStruct((n_nodes, 64), jnp.float32),
    )(x, agg, gp['W1'], gp['b1'].reshape(1, 64), gp['W2'],
      gp['b2'].reshape(1, 64), lp['g'].reshape(1, 64), lp['b'].reshape(1, 64),
      gp['eps'].reshape(1, 1))


def _contrast_body(spc_ref, clade_ref, out_ref):
    spc = spc_ref[...]
    clade = clade_ref[...]
    outm = 1.0 - clade
    validf = (spc > 0).astype(jnp.float32)
    dup = (spc > 1).astype(jnp.float32)

    dn = (((1,), (1,)), ((), ()))

    def mm(a, b):
        return lax.dot_general(a, b, dn, preferred_element_type=jnp.float32)

    cb = mm(spc, clade)
    co = mm(spc, outm)
    vb = mm(validf, clade)
    vo = mm(validf, outm)
    db = mm(dup, clade)
    do = mm(dup, outm)

    has = (vb > 0) & (vo > 0)
    avg_b = cb / jnp.maximum(vb, 1.0)
    avg_o = co / jnp.maximum(vo, 1.0)
    cr = avg_b / jnp.maximum(avg_o, 0.1)
    fdb = db / jnp.maximum(vb, 1.0)
    fdo = do / jnp.maximum(vo, 1.0)
    dc = fdb - fdo
    m = has.astype(jnp.float32)
    n = m.sum(axis=0)

    feats = []
    for xx in (avg_b, cr, fdb, fdo, dc):
        mu = (xx * m).sum(0) / jnp.maximum(n, 1.0)
        var = (((xx - mu[None, :]) ** 2) * m).sum(0) / jnp.maximum(n - 1.0, 1.0)
        sd = jnp.where(n > 1, jnp.sqrt(jnp.maximum(var, 0.0) + 1e-12), 0.0)
        feats.append(mu[:, None])
        feats.append(sd[:, None])
    contrast = jnp.concatenate(feats, axis=1)
    edge_ok = (clade.sum(1) > 0) & (outm.sum(1) > 0) & (n > 0)
    out_ref[...] = contrast * edge_ok[:, None].astype(jnp.float32)


def _contrast(spc2d, clade_f):
    e = clade_f.shape[0]
    return pl.pallas_call(
        _contrast_body,
        out_shape=jax.ShapeDtypeStruct((e, 10), jnp.float32),
    )(spc2d, clade_f)


def _ginstat_body(pool_ref, cnt_ref, clade_ref, out_ref, m1_ref, m2_ref,
                  c_ref):
    i = pl.program_id(0)
    nsteps = pl.num_programs(0)

    @pl.when(i == 0)
    def _():
        m1_ref[...] = jnp.zeros_like(m1_ref)
        m2_ref[...] = jnp.zeros_like(m2_ref)
        c_ref[...] = jnp.zeros_like(c_ref)

    p = pool_ref[...]
    c = cnt_ref[0]
    v = (c > 0).astype(jnp.float32)
    mp = p / jnp.maximum(c, 1.0)[:, :, None]
    mpv = mp * v[:, :, None]
    m1_ref[...] += mpv.sum(axis=0)
    m2_ref[...] += (mpv * mp).sum(axis=0)
    c_ref[...] += v.sum(axis=0)[None, :]

    @pl.when(i == nsteps - 1)
    def _():
        clade = clade_ref[...]
        dn = (((1,), (0,)), ((), ()))

        def mm(a, b):
            return lax.dot_general(a, b, dn,
                                   preferred_element_type=jnp.float32)

        s1 = mm(clade, m1_ref[...])
        s2 = mm(clade, m2_ref[...])
        ne = (clade * c_ref[0][None, :]).sum(axis=1, keepdims=True)
        nec = jnp.maximum(ne, 1.0)
        mean_e = s1 / nec
        var_e = (s2 - nec * mean_e ** 2) / jnp.maximum(ne - 1.0, 1.0)
        std_e = jnp.where(ne > 1,
                          jnp.sqrt(jnp.maximum(var_e, 0.0) + 1e-12), 0.0)
        out_ref[...] = jnp.concatenate([mean_e, std_e], axis=1)


def _ginstat(pool3, cnt3, clade_f):
    e = clade_f.shape[0]
    grid = N_GT // GT_BLK
    return pl.pallas_call(
        _ginstat_body,
        grid=(grid,),
        in_specs=[
            pl.BlockSpec((GT_BLK, N_SPECIES, 64), lambda i: (i, 0, 0)),
            pl.BlockSpec((1, GT_BLK, N_SPECIES), lambda i: (i, 0, 0)),
            pl.BlockSpec((e, N_SPECIES), lambda i: (0, 0)),
        ],
        out_specs=pl.BlockSpec((e, 128), lambda i: (0, 0)),
        out_shape=jax.ShapeDtypeStruct((e, 128), jnp.float32),
        scratch_shapes=[
            pltpu.VMEM((N_SPECIES, 64), jnp.float32),
            pltpu.VMEM((N_SPECIES, 64), jnp.float32),
            pltpu.VMEM((1, N_SPECIES), jnp.float32),
        ],
    )(pool3, cnt3, clade_f)


N_NODES_C = 400000
TREES_PER_W = 16
PCHUNK = 1024
PCHUNK_PAD = PCHUNK + 8
BINS_W = 401 * 64


def _sload(ref, i):
    return ref[pl.ds(i, 16)][0]


def _pool_body(code_hbm, x_hbm, starts_hbm, zeros_hbm, oh_hbm,
               pool_hbm, cnt_hbm,
               starts_v, code_v, x_v, bins_v, cnt16_v, oh_v):
    c = lax.axis_index("c")
    s = lax.axis_index("s")
    wid = s * 2 + c
    pltpu.sync_copy(starts_hbm, starts_v)
    pltpu.sync_copy(oh_hbm, oh_v)
    t0 = jnp.minimum(wid * TREES_PER_W, N_GT)
    t1 = jnp.minimum(t0 + TREES_PER_W, N_GT)

    def tree_body(t, carry):
        pltpu.sync_copy(zeros_hbm.at[pl.ds(0, BINS_W)], bins_v)
        pltpu.sync_copy(zeros_hbm.at[pl.ds(0, 416 * 16)], cnt16_v)
        n0 = _sload(starts_v, t)
        n1 = _sload(starts_v, t + 1)
        nch = (n1 - n0 + PCHUNK - 1) // PCHUNK

        def chunk_body(k, carry2):
            pos = n0 + k * PCHUNK
            m = jnp.minimum(PCHUNK, n1 - pos)
            posc = jnp.minimum((pos // 8) * 8, N_NODES_C - PCHUNK_PAD)
            off = pos - posc
            pltpu.sync_copy(code_hbm.at[pl.ds(posc, PCHUNK_PAD)],
                            code_v.at[pl.ds(0, PCHUNK_PAD)])
            pltpu.sync_copy(x_hbm.at[pl.ds(posc * 64, PCHUNK_PAD * 64)], x_v)

            def node_body(i, carry3):
                sp_i = _sload(code_v, off + i)
                base = sp_i * 64
                xb = (off + i) * 64
                for j in range(4):
                    r = x_v[pl.ds(xb + 16 * j, 16)]
                    b = bins_v[pl.ds(base + 16 * j, 16)]
                    bins_v[pl.ds(base + 16 * j, 16)] = b + r
                one0 = oh_v[pl.ds(0, 16)]
                cb = cnt16_v[pl.ds(sp_i * 16, 16)]
                cnt16_v[pl.ds(sp_i * 16, 16)] = cb + one0
                return carry3

            lax.fori_loop(0, m, node_body, 0)
            return carry2

        lax.fori_loop(0, nch, chunk_body, 0)
        pltpu.sync_copy(bins_v.at[pl.ds(0, N_SPECIES * 64)],
                        pool_hbm.at[pl.ds(t * (N_SPECIES * 64),
                                          N_SPECIES * 64)])
        pltpu.sync_copy(cnt16_v.at[pl.ds(0, N_SPECIES * 16)],
                        cnt_hbm.at[pl.ds(t * (N_SPECIES * 16),
                                         N_SPECIES * 16)])
        return carry

    lax.fori_loop(t0, t1, tree_body, 0)


def _sc_pool(code, x_flat, starts_pad, zeros, onehot):
    f = pl.kernel(
        _pool_body,
        out_type=[
            jax.ShapeDtypeStruct((N_GT * N_SPECIES * 64,), jnp.float32),
            jax.ShapeDtypeStruct((N_GT * N_SPECIES * 16,), jnp.float32),
        ],
        mesh=plsc.VectorSubcoreMesh(core_axis_name="c", subcore_axis_name="s",
                                    num_cores=2, num_subcores=16),
        scratch_types=[
            pltpu.VMEM((528,), jnp.int32),
            pltpu.VMEM((PCHUNK_PAD + 16,), jnp.int32),
            pltpu.VMEM((PCHUNK_PAD * 64,), jnp.float32),
            pltpu.VMEM((BINS_W,), jnp.float32),
            pltpu.VMEM((416 * 16,), jnp.float32),
            pltpu.VMEM((16,), jnp.float32),
        ],
    )
    return f(code, x_flat, starts_pad, zeros, onehot)


A_RCH = 20000
A_NCHP = 10
A_BE = 10000
A_NSB = 5
A_EPT = 50000
A_PADROWS = 480
ZR = 320


def _agg_body(src_hbm, dst_hbm, x_hbm, z2_hbm, agg_hbm,
              dst_v, src_v, cs_v, cd_v, rows_v, zrow_v, accum_sh, gsem):
    c = lax.axis_index("c")
    s = lax.axis_index("s")
    pltpu.sync_copy(z2_hbm, zrow_v)

    for ci in range(A_NCHP):
        chunk = 2 * ci + c
        lo = chunk * A_RCH
        for zz in range(4):
            pltpu.sync_copy(zrow_v,
                            accum_sh.at[pl.ds(s * (4 * ZR) + zz * ZR, ZR)])
        plsc.subcore_barrier()

        for b in range(A_NSB):
            eoff = s * A_EPT + b * A_BE
            pltpu.sync_copy(dst_hbm.at[pl.ds(eoff, A_BE)], dst_v)
            pltpu.sync_copy(src_hbm.at[pl.ds(eoff, A_BE)], src_v)

            def scan_body(i, kk):
                d = dst_v[pl.ds(16 * i, 16)]
                sv = src_v[pl.ds(16 * i, 16)]
                lo_v = jnp.full((16,), lo, jnp.int32)
                hi_v = jnp.full((16,), lo + A_RCH, jnp.int32)
                m_in = (d >= lo_v) & (d < hi_v)
                mi = m_in.astype(jnp.int32)
                pos = (jnp.full((16,), kk, jnp.int32)
                       + plsc.cumsum(mi) - mi)
                trash = jnp.full((16,), A_BE + 144, jnp.int32)
                idx = jnp.where(m_in, pos, trash)
                plsc.store_scatter(cs_v, [idx], sv)
                plsc.store_scatter(cd_v, [idx], d - lo_v)
                return kk + jnp.sum(mi)

            kk = lax.fori_loop(0, A_BE // 16, scan_body, 0)

            zi = jnp.zeros((16,), jnp.int32)
            di = jnp.full((16,), A_RCH, jnp.int32)
            for j in range(8):
                cs_v[pl.ds(kk + 16 * j, 16)] = zi
                cd_v[pl.ds(kk + 16 * j, 16)] = di
            nb = (kk + 127) // 128

            def batch_body(g, carry):
                pltpu.async_copy(x_hbm.at[cs_v.at[pl.ds(g * 128, 128)]],
                                 rows_v, gsem).wait()
                for q in range(8):
                    idx = cd_v[pl.ds(g * 128 + 16 * q, 16)]
                    pltpu.sync_copy(rows_v.at[pl.ds(16 * q, 16)],
                                    accum_sh.at[idx], add=True)
                return carry

            lax.fori_loop(0, nb, batch_body, 0)

        plsc.subcore_barrier()
        pltpu.sync_copy(accum_sh.at[pl.ds(s * 1248, 1248)],
                        agg_hbm.at[pl.ds(lo + s * 1248, 1248)])

        @pl.when(s == 15)
        def _():
            pltpu.sync_copy(accum_sh.at[pl.ds(19968, 32)],
                            agg_hbm.at[pl.ds(lo + 19968, 32)])

        plsc.subcore_barrier()


def _sc_agg(src, dst, x2d, z2):
    f = pl.kernel(
        _agg_body,
        out_type=jax.ShapeDtypeStruct((N_NODES_C, 64), jnp.float32),
        mesh=plsc.VectorSubcoreMesh(core_axis_name="c", subcore_axis_name="s",
                                    num_cores=2, num_subcores=16),
        scratch_types=[
            pltpu.VMEM((A_BE,), jnp.int32),
            pltpu.VMEM((A_BE,), jnp.int32),
            pltpu.VMEM((A_BE + 160,), jnp.int32),
            pltpu.VMEM((A_BE + 160,), jnp.int32),
            pltpu.VMEM((128, 64), jnp.float32),
            pltpu.VMEM((ZR, 64), jnp.float32),
            pltpu.VMEM_SHARED((A_RCH + A_PADROWS, 64), jnp.float32),
            pltpu.SemaphoreType.DMA,
        ],
    )
    return f(src, dst, x2d, z2)


def kernel(species_emb, gin_params, ln_params, species_ids, leaf_mask,
           batch_ids, edge_index, clade_mask, n_edges):
    n_nodes = species_ids.shape[0]
    sp = species_ids
    valid = leaf_mask & (sp >= 0)
    code = jnp.where(valid, jnp.clip(sp, 0, N_SPECIES - 1),
                     N_SPECIES).astype(jnp.int32)
    clade = clade_mask.astype(jnp.float32)

    emb_ids = jnp.where(sp < 0, N_SPECIES, sp)
    emb_ids = jnp.clip(emb_ids, 0, N_SPECIES).astype(jnp.int32)
    table_pad = jnp.zeros((EMB_PAD, 64), jnp.float32).at[:N_SPECIES + 1].set(
        species_emb)
    x = _emb_lookup(emb_ids, table_pad, n_nodes)
    src = edge_index[0]
    dst = edge_index[1]
    for gp, lp in zip(gin_params, ln_params):
        agg = jnp.zeros_like(x).at[dst].add(x[src])
        x = _mlp_layer(x, agg, gp, lp, n_nodes)

    starts = jnp.searchsorted(
        batch_ids, jnp.arange(N_GT + 1, dtype=batch_ids.dtype),
        side='left').astype(jnp.int32)
    starts_pad = jnp.concatenate(
        [starts, jnp.full((528 - N_GT - 1,), n_nodes, jnp.int32)])
    zeros = jnp.zeros((BINS_W,), jnp.float32)
    onehot = jnp.zeros((16,), jnp.float32).at[0].set(1.0)
    pool_flat, cnt16 = _sc_pool(code, x.reshape(-1), starts_pad, zeros,
                                onehot)
    sp_count = cnt16.reshape(N_GT * N_SPECIES, 16)[:, 0]

    sp_count_2d = sp_count.reshape(N_GT, N_SPECIES)
    contrast = _contrast(sp_count_2d, clade)

    pool3 = pool_flat.reshape(N_GT, N_SPECIES, 64)
    cnt3 = sp_count.reshape(N_GT // GT_BLK, GT_BLK, N_SPECIES)
    gin_feats = _ginstat(pool3, cnt3, clade)

    return jnp.concatenate([gin_feats, contrast], axis=1)


# speedup vs baseline: 1.2955x; 1.0792x over previous
import functools

import jax
import jax.numpy as jnp
from jax import lax
from jax.experimental import pallas as pl
from jax.experimental.pallas import tpu as pltpu
from jax.experimental.pallas import tpu_sc as plsc

N_SPECIES = 400
N_GT = 500
EMB_PAD = 512
ROW_BLK = 2000
GT_BLK = 2


def _emb_body(ids_ref, table_ref, out_ref):
    ids = ids_ref[0, 0]
    table = table_ref[...]
    cols = lax.broadcasted_iota(jnp.int32, (ROW_BLK, EMB_PAD), 1)
    onehot = (ids[:, None] == cols).astype(jnp.float32)
    out_ref[...] = lax.dot_general(
        onehot, table, (((1,), (0,)), ((), ())),
        preferred_element_type=jnp.float32)


def _emb_lookup(emb_ids, table_pad, n_nodes):
    grid = n_nodes // ROW_BLK
    ids3 = emb_ids.reshape(grid, 1, ROW_BLK)
    return pl.pallas_call(
        _emb_body,
        grid=(grid,),
        in_specs=[
            pl.BlockSpec((1, 1, ROW_BLK), lambda i: (i, 0, 0)),
            pl.BlockSpec((EMB_PAD, 64), lambda i: (0, 0)),
        ],
        out_specs=pl.BlockSpec((ROW_BLK, 64), lambda i: (i, 0)),
        out_shape=jax.ShapeDtypeStruct((n_nodes, 64), jnp.float32),
    )(ids3, table_pad)


def _mlp_body(x_ref, agg_ref, w1_ref, b1_ref, w2_ref, b2_ref, g_ref, b_ref,
              eps_ref, out_ref):
    x = x_ref[...]
    agg = agg_ref[...]
    eps = eps_ref[0, 0]
    h = (1.0 + eps) * x + agg
    dn = (((1,), (0,)), ((), ()))
    z = lax.dot_general(h, w1_ref[...], dn, preferred_element_type=jnp.float32)
    z = jnp.maximum(z + b1_ref[0][None, :], 0.0)
    h2 = lax.dot_general(z, w2_ref[...], dn, preferred_element_type=jnp.float32)
    h2 = h2 + b2_ref[0][None, :]
    xn = x + h2
    mu = jnp.mean(xn, axis=-1, keepdims=True)
    var = jnp.mean((xn - mu) ** 2, axis=-1, keepdims=True)
    out_ref[...] = ((xn - mu) * lax.rsqrt(var + 1e-5) * g_ref[0][None, :]
                    + b_ref[0][None, :])


def _mlp_layer(x, agg, gp, lp, n_nodes):
    grid = n_nodes // ROW_BLK
    row = pl.BlockSpec((ROW_BLK, 64), lambda i: (i, 0))
    mat = pl.BlockSpec((64, 64), lambda i: (0, 0))
    vec = pl.BlockSpec((1, 64), lambda i: (0, 0))
    scl = pl.BlockSpec((1, 1), lambda i: (0, 0))
    return pl.pallas_call(
        _mlp_body,
        grid=(grid,),
        in_specs=[row, row, mat, vec, mat, vec, vec, vec, scl],
        out_specs=row,
        out_shape=jax.ShapeDtypeStruct((n_nodes, 64), jnp.float32),
    )(x, agg, gp['W1'], gp['b1'].reshape(1, 64), gp['W2'],
      gp['b2'].reshape(1, 64), lp['g'].reshape(1, 64), lp['b'].reshape(1, 64),
      gp['eps'].reshape(1, 1))


def _contrast_body(spc_ref, clade_ref, out_ref):
    spc = spc_ref[...]
    clade = clade_ref[...]
    outm = 1.0 - clade
    validf = (spc > 0).astype(jnp.float32)
    dup = (spc > 1).astype(jnp.float32)

    dn = (((1,), (1,)), ((), ()))

    def mm(a, b):
        return lax.dot_general(a, b, dn, preferred_element_type=jnp.float32)

    cb = mm(spc, clade)
    co = mm(spc, outm)
    vb = mm(validf, clade)
    vo = mm(validf, outm)
    db = mm(dup, clade)
    do = mm(dup, outm)

    has = (vb > 0) & (vo > 0)
    avg_b = cb / jnp.maximum(vb, 1.0)
    avg_o = co / jnp.maximum(vo, 1.0)
    cr = avg_b / jnp.maximum(avg_o, 0.1)
    fdb = db / jnp.maximum(vb, 1.0)
    fdo = do / jnp.maximum(vo, 1.0)
    dc = fdb - fdo
    m = has.astype(jnp.float32)
    n = m.sum(axis=0)

    feats = []
    for xx in (avg_b, cr, fdb, fdo, dc):
        mu = (xx * m).sum(0) / jnp.maximum(n, 1.0)
        var = (((xx - mu[None, :]) ** 2) * m).sum(0) / jnp.maximum(n - 1.0, 1.0)
        sd = jnp.where(n > 1, jnp.sqrt(jnp.maximum(var, 0.0) + 1e-12), 0.0)
        feats.append(mu[:, None])
        feats.append(sd[:, None])
    contrast = jnp.concatenate(feats, axis=1)
    edge_ok = (clade.sum(1) > 0) & (outm.sum(1) > 0) & (n > 0)
    out_ref[...] = contrast * edge_ok[:, None].astype(jnp.float32)


def _contrast(spc2d, clade_f):
    e = clade_f.shape[0]
    return pl.pallas_call(
        _contrast_body,
        out_shape=jax.ShapeDtypeStruct((e, 10), jnp.float32),
    )(spc2d, clade_f)


def _ginstat_body(pool_ref, cnt_ref, clade_ref, out_ref, m1_ref, m2_ref,
                  c_ref):
    i = pl.program_id(0)
    nsteps = pl.num_programs(0)

    @pl.when(i == 0)
    def _():
        m1_ref[...] = jnp.zeros_like(m1_ref)
        m2_ref[...] = jnp.zeros_like(m2_ref)
        c_ref[...] = jnp.zeros_like(c_ref)

    p = pool_ref[...]
    c = cnt_ref[0]
    v = (c > 0).astype(jnp.float32)
    mp = p / jnp.maximum(c, 1.0)[:, :, None]
    mpv = mp * v[:, :, None]
    m1_ref[...] += mpv.sum(axis=0)
    m2_ref[...] += (mpv * mp).sum(axis=0)
    c_ref[...] += v.sum(axis=0)[None, :]

    @pl.when(i == nsteps - 1)
    def _():
        clade = clade_ref[...]
        dn = (((1,), (0,)), ((), ()))

        def mm(a, b):
            return lax.dot_general(a, b, dn,
                                   preferred_element_type=jnp.float32)

        s1 = mm(clade, m1_ref[...])
        s2 = mm(clade, m2_ref[...])
        ne = (clade * c_ref[0][None, :]).sum(axis=1, keepdims=True)
        nec = jnp.maximum(ne, 1.0)
        mean_e = s1 / nec
        var_e = (s2 - nec * mean_e ** 2) / jnp.maximum(ne - 1.0, 1.0)
        std_e = jnp.where(ne > 1,
                          jnp.sqrt(jnp.maximum(var_e, 0.0) + 1e-12), 0.0)
        out_ref[...] = jnp.concatenate([mean_e, std_e], axis=1)


def _ginstat(pool3, cnt3, clade_f):
    e = clade_f.shape[0]
    grid = N_GT // GT_BLK
    return pl.pallas_call(
        _ginstat_body,
        grid=(grid,),
        in_specs=[
            pl.BlockSpec((GT_BLK, N_SPECIES, 64), lambda i: (i, 0, 0)),
            pl.BlockSpec((1, GT_BLK, N_SPECIES), lambda i: (i, 0, 0)),
            pl.BlockSpec((e, N_SPECIES), lambda i: (0, 0)),
        ],
        out_specs=pl.BlockSpec((e, 128), lambda i: (0, 0)),
        out_shape=jax.ShapeDtypeStruct((e, 128), jnp.float32),
        scratch_shapes=[
            pltpu.VMEM((N_SPECIES, 64), jnp.float32),
            pltpu.VMEM((N_SPECIES, 64), jnp.float32),
            pltpu.VMEM((1, N_SPECIES), jnp.float32),
        ],
    )(pool3, cnt3, clade_f)


N_NODES_C = 400000
TREES_PER_W = 16
PCHUNK = 1024
PCHUNK_PAD = PCHUNK + 8
XSTR = 64
BINS_W = 401 * 64


def _sload(ref, i):
    return ref[pl.ds(i, 16)][0]


def _pool_body(code_hbm, x_hbm, starts_hbm, zeros_hbm, oh_hbm,
               pool_hbm, cnt_hbm,
               starts_v, code_v, x_v, bins_v, cnt16_v, oh_v):
    c = lax.axis_index("c")
    s = lax.axis_index("s")
    wid = s * 2 + c
    pltpu.sync_copy(starts_hbm, starts_v)
    pltpu.sync_copy(oh_hbm, oh_v)
    t0 = jnp.minimum(wid * TREES_PER_W, N_GT)
    t1 = jnp.minimum(t0 + TREES_PER_W, N_GT)

    def tree_body(t, carry):
        pltpu.sync_copy(zeros_hbm.at[pl.ds(0, BINS_W)], bins_v)
        pltpu.sync_copy(zeros_hbm.at[pl.ds(0, 416 * 16)], cnt16_v)
        n0 = _sload(starts_v, t)
        n1 = _sload(starts_v, t + 1)
        nch = (n1 - n0 + PCHUNK - 1) // PCHUNK

        def chunk_body(k, carry2):
            pos = n0 + k * PCHUNK
            m = jnp.minimum(PCHUNK, n1 - pos)
            posc = jnp.minimum((pos // 8) * 8, N_NODES_C - PCHUNK_PAD)
            off = pos - posc
            pltpu.sync_copy(code_hbm.at[pl.ds(posc, PCHUNK_PAD)],
                            code_v.at[pl.ds(0, PCHUNK_PAD)])
            pltpu.sync_copy(x_hbm.at[pl.ds(posc * XSTR, PCHUNK_PAD * XSTR)],
                            x_v)

            def node_body(i, carry3):
                sp_i = _sload(code_v, off + i)
                base = sp_i * 64
                xb = (off + i) * XSTR
                for j in range(4):
                    r = x_v[pl.ds(xb + 16 * j, 16)]
                    b = bins_v[pl.ds(base + 16 * j, 16)]
                    bins_v[pl.ds(base + 16 * j, 16)] = b + r
                one0 = oh_v[pl.ds(0, 16)]
                cb = cnt16_v[pl.ds(sp_i * 16, 16)]
                cnt16_v[pl.ds(sp_i * 16, 16)] = cb + one0
                return carry3

            lax.fori_loop(0, m, node_body, 0)
            return carry2

        lax.fori_loop(0, nch, chunk_body, 0)
        pltpu.sync_copy(bins_v.at[pl.ds(0, N_SPECIES * 64)],
                        pool_hbm.at[pl.ds(t * (N_SPECIES * 64),
                                          N_SPECIES * 64)])
        pltpu.sync_copy(cnt16_v.at[pl.ds(0, N_SPECIES * 16)],
                        cnt_hbm.at[pl.ds(t * (N_SPECIES * 16),
                                         N_SPECIES * 16)])
        return carry

    lax.fori_loop(t0, t1, tree_body, 0)


def _sc_pool(code, x_flat, starts_pad, zeros, onehot):
    f = pl.kernel(
        _pool_body,
        out_type=[
            jax.ShapeDtypeStruct((N_GT * N_SPECIES * 64,), jnp.float32),
            jax.ShapeDtypeStruct((N_GT * N_SPECIES * 16,), jnp.float32),
        ],
        mesh=plsc.VectorSubcoreMesh(core_axis_name="c", subcore_axis_name="s",
                                    num_cores=2, num_subcores=16),
        scratch_types=[
            pltpu.VMEM((528,), jnp.int32),
            pltpu.VMEM((PCHUNK_PAD + 16,), jnp.int32),
            pltpu.VMEM((PCHUNK_PAD * XSTR,), jnp.float32),
            pltpu.VMEM((BINS_W,), jnp.float32),
            pltpu.VMEM((416 * 16,), jnp.float32),
            pltpu.VMEM((16,), jnp.float32),
        ],
    )
    return f(code, x_flat, starts_pad, zeros, onehot)


def kernel(species_emb, gin_params, ln_params, species_ids, leaf_mask,
           batch_ids, edge_index, clade_mask, n_edges):
    n_nodes = species_ids.shape[0]
    sp = species_ids
    valid = leaf_mask & (sp >= 0)
    code = jnp.where(valid, jnp.clip(sp, 0, N_SPECIES - 1),
                     N_SPECIES).astype(jnp.int32)
    clade = clade_mask.astype(jnp.float32)

    emb_ids = jnp.where(sp < 0, N_SPECIES, sp)
    emb_ids = jnp.clip(emb_ids, 0, N_SPECIES).astype(jnp.int32)
    table_pad = jnp.zeros((EMB_PAD, 64), jnp.float32).at[:N_SPECIES + 1].set(
        species_emb)
    x = _emb_lookup(emb_ids, table_pad, n_nodes)
    e_src = edge_index[0]
    e_dst = edge_index[1]
    for gp, lp in zip(gin_params, ln_params):
        agg = jnp.zeros_like(x).at[e_dst].add(x[e_src])
        x = _mlp_layer(x, agg, gp, lp, n_nodes)

    starts = jnp.searchsorted(
        batch_ids, jnp.arange(N_GT + 1, dtype=batch_ids.dtype),
        side='left').astype(jnp.int32)
    starts_pad = jnp.concatenate(
        [starts, jnp.full((528 - N_GT - 1,), n_nodes, jnp.int32)])
    zeros = jnp.zeros((BINS_W,), jnp.float32)
    onehot = jnp.zeros((16,), jnp.float32).at[0].set(1.0)
    pool_flat, cnt16 = _sc_pool(code, x.reshape(-1), starts_pad, zeros,
                                onehot)
    sp_count = cnt16.reshape(N_GT * N_SPECIES, 16)[:, 0]

    sp_count_2d = sp_count.reshape(N_GT, N_SPECIES)
    contrast = _contrast(sp_count_2d, clade)

    pool3 = pool_flat.reshape(N_GT, N_SPECIES, 64)
    cnt3 = sp_count.reshape(N_GT // GT_BLK, GT_BLK, N_SPECIES)
    gin_feats = _ginstat(pool3, cnt3, clade)

    return jnp.concatenate([gin_feats, contrast], axis=1)
